# Initial kernel scaffold; baseline (speedup 1.0000x reference)
import jax
import jax.numpy as jnp
from jax.experimental import pallas as pl


def kernel(x, W1, b1, W2, b2, Wr, br, Ew1, Eb1, Ew3, Eb3, Ew2, Eb2):
    raise NotImplementedError("write your pallas kernel here")


# speedup vs baseline: 1.6312x; 1.6312x over previous
import functools

import jax
import jax.numpy as jnp
from jax.experimental import pallas as pl
from jax.experimental.pallas import tpu as pltpu

DIM = 1024
INNER = 4096
E = 8
K = 2
ALPHA = 0.5
T = 2048
CAP = 1024

ROW_BLK = 256
J_BLK = 1024


def _dense_body(x_ref, w1_ref, b1_ref, w2_ref, b2_ref, o_ref):
    h = jnp.dot(x_ref[...], w1_ref[...], preferred_element_type=jnp.float32)
    h = jax.nn.gelu(h + b1_ref[...])
    o_ref[...] = jnp.dot(h, w2_ref[...], preferred_element_type=jnp.float32) + b2_ref[...]


def _dense_branch(x, W1, b1, W2, b2):
    grid = (T // ROW_BLK,)
    return pl.pallas_call(
        _dense_body,
        grid=grid,
        in_specs=[
            pl.BlockSpec((ROW_BLK, DIM), lambda i: (i, 0)),
            pl.BlockSpec((DIM, INNER), lambda i: (0, 0)),
            pl.BlockSpec((1, INNER), lambda i: (0, 0)),
            pl.BlockSpec((INNER, DIM), lambda i: (0, 0)),
            pl.BlockSpec((1, DIM), lambda i: (0, 0)),
        ],
        out_specs=pl.BlockSpec((ROW_BLK, DIM), lambda i: (i, 0)),
        out_shape=jax.ShapeDtypeStruct((T, DIM), jnp.float32),
    )(x, W1, b1.reshape(1, INNER), W2, b2.reshape(1, DIM))


def _expert_body(xd_ref, w1_ref, b1_ref, w3_ref, b3_ref, w2_ref, b2_ref, o_ref):
    j = pl.program_id(1)
    x = xd_ref[...]
    h1 = jnp.dot(x, w1_ref[0], preferred_element_type=jnp.float32) + b1_ref[0]
    h3 = jnp.dot(x, w3_ref[0], preferred_element_type=jnp.float32) + b3_ref[0]
    h = jax.nn.silu(h1) * h3
    part = jnp.dot(h, w2_ref[0], preferred_element_type=jnp.float32)

    @pl.when(j == 0)
    def _init():
        o_ref[...] = part + b2_ref[0]

    @pl.when(j > 0)
    def _acc():
        o_ref[...] += part


def _expert_ffn(Xd, Ew1, Eb1, Ew3, Eb3, Ew2, Eb2):
    nj = INNER // J_BLK
    grid = (E, nj)
    return pl.pallas_call(
        _expert_body,
        grid=grid,
        in_specs=[
            pl.BlockSpec((CAP, DIM), lambda e, j: (e, 0)),
            pl.BlockSpec((1, DIM, J_BLK), lambda e, j: (e, 0, j)),
            pl.BlockSpec((1, 1, J_BLK), lambda e, j: (e, 0, j)),
            pl.BlockSpec((1, DIM, J_BLK), lambda e, j: (e, 0, j)),
            pl.BlockSpec((1, 1, J_BLK), lambda e, j: (e, 0, j)),
            pl.BlockSpec((1, J_BLK, DIM), lambda e, j: (e, j, 0)),
            pl.BlockSpec((1, 1, DIM), lambda e, j: (e, 0, 0)),
        ],
        out_specs=pl.BlockSpec((CAP, DIM), lambda e, j: (e, 0)),
        out_shape=jax.ShapeDtypeStruct((E * CAP, DIM), jnp.float32),
    )(Xd.reshape(E * CAP, DIM), Ew1, Eb1.reshape(E, 1, INNER), Ew3,
      Eb3.reshape(E, 1, INNER), Ew2, Eb2.reshape(E, 1, DIM))


def kernel(x, W1, b1, W2, b2, Wr, br, Ew1, Eb1, Ew3, Eb3, Ew2, Eb2):
    y_dense = _dense_branch(x, W1, b1, W2, b2)

    logits = x @ Wr + br
    probs = jax.nn.softmax(logits, axis=-1)
    gate_vals, expert_idx = jax.lax.top_k(probs, K)
    gate_vals = gate_vals / jnp.sum(gate_vals, axis=-1, keepdims=True)

    n_slots = T * K
    flat_eid = expert_idx.reshape(-1)
    flat_gate = gate_vals.reshape(-1)

    onehot = jax.nn.one_hot(flat_eid, E, dtype=jnp.int32)
    ranks_all = jnp.cumsum(onehot, axis=0) - onehot
    rank = jnp.take_along_axis(ranks_all, flat_eid[:, None], axis=1)[:, 0]
    valid = rank < CAP
    row = flat_eid * CAP + jnp.minimum(rank, CAP - 1)
    tok = jnp.arange(n_slots, dtype=jnp.int32) // K

    tok_for_row = jnp.zeros((E * CAP,), jnp.int32).at[jnp.where(valid, row, E * CAP)].set(
        tok, mode="drop")
    Xd = x[tok_for_row]

    Ye = _expert_ffn(Xd, Ew1, Eb1, Ew3, Eb3, Ew2, Eb2)

    w = flat_gate * valid.astype(x.dtype)
    y_moe = jnp.zeros_like(x).at[tok].add(w[:, None] * Ye[row])

    me = jnp.mean(probs, axis=0)
    ce = jnp.mean(jnp.sum(jax.nn.one_hot(expert_idx, E, dtype=x.dtype), axis=1), axis=0) / K
    aux = E * jnp.sum(me * ce)

    y = ALPHA * y_dense + (1.0 - ALPHA) * y_moe
    return (y, aux)
